# baseline (speedup 1.0000x reference)
import jax
import jax.numpy as jnp
from jax.experimental import pallas as pl
from jax.experimental.pallas import tpu as pltpu

_N = 20000
_CLS1 = 81
_NCH = 10
_CH = _N // _NCH
_MAXO = 100
_THC = 0.5
_THI = 0.5
_VAR = 0.125


def _dpp_body(conf_ref, loc_ref, anc_ref,
              x1_ref, y1_ref, x2_ref, y2_ref, sc_ref, qs_ref):
    f32 = jnp.float32
    zeros_out = jnp.zeros((1, 80, _MAXO), f32)
    x1_ref[...] = zeros_out
    y1_ref[...] = zeros_out
    x2_ref[...] = zeros_out
    y2_ref[...] = zeros_out
    sc_ref[...] = zeros_out

    for j in range(_NCH):
        lo, hi = _CH * j, _CH * (j + 1)
        x = conf_ref[0, lo:hi, :]
        m = jnp.max(x, axis=1, keepdims=True)
        denom = jnp.sum(jnp.exp(x - m), axis=1, keepdims=True)
        score = 1.0 / denom
        score = jnp.where(score >= _THC, score, 0.0)
        qs_ref[:, j:j + 1] = score

    qs0 = qs_ref[:, 0:_NCH]
    rowi = jax.lax.broadcasted_iota(jnp.int32, (_CH, _NCH), 0)
    lanei = jax.lax.broadcasted_iota(jnp.int32, (_CH, _NCH), 1)
    flatn = lanei * _CH + rowi
    l128 = jax.lax.broadcasted_iota(jnp.int32, (1, 128), 1)
    l100 = jax.lax.broadcasted_iota(jnp.int32, (1, _MAXO), 1)
    l81 = jax.lax.broadcasted_iota(jnp.int32, (1, _CLS1), 1)
    l32 = jax.lax.broadcasted_iota(jnp.int32, (1, 32), 1)

    n_cand = jnp.sum((qs0 > 0.0).astype(jnp.int32))

    def body(_, carry):
        qs, counters = carry
        mx = jnp.max(qs)
        sel = jnp.min(jnp.where(qs == mx, flatn, _N))
        qs = jnp.where(flatn == sel, 0.0, qs)

        crow = conf_ref[0, pl.ds(sel, 1), :]
        cm = jnp.max(crow)
        cls = jnp.min(jnp.where(crow == cm, l81, _CLS1))
        valid = cls >= 1
        ci = jnp.where(valid, cls - 1, 0)

        lrow = loc_ref[0, pl.ds(sel // 8, 1), :]
        arow = anc_ref[pl.ds(sel // 8, 1), :]
        base = (sel % 8) * 4
        gl = lambda v, k: jnp.sum(jnp.where(l32 == base + k, v, 0.0))
        dl = lrow * _VAR
        de = jnp.exp(dl)
        a0, a1, a2, a3 = (gl(arow, k) for k in range(4))
        cx = gl(dl, 0) * a2 + a0
        cy = gl(dl, 1) * a3 + a1
        w = gl(de, 2) * a2
        h = gl(de, 3) * a3
        cx1, cy1 = cx - w / 2.0, cy - h / 2.0
        cx2, cy2 = cx + w / 2.0, cy + h / 2.0

        r = jnp.sum(jnp.where(l128 == ci, counters, 0.0)
                    ).astype(jnp.int32)
        counters = counters + jnp.where(valid & (l128 == ci), 1.0, 0.0)

        kx1 = x1_ref[0, pl.ds(ci, 1), :]
        ky1 = y1_ref[0, pl.ds(ci, 1), :]
        kx2 = x2_ref[0, pl.ds(ci, 1), :]
        ky2 = y2_ref[0, pl.ds(ci, 1), :]
        ks = sc_ref[0, pl.ds(ci, 1), :]
        iw = jnp.maximum(jnp.minimum(cx2, kx2) - jnp.maximum(cx1, kx1), 0.0)
        ih = jnp.maximum(jnp.minimum(cy2, ky2) - jnp.maximum(cy1, ky1), 0.0)
        inter = iw * ih
        a_c = (cx2 - cx1) * (cy2 - cy1)
        a_k = (kx2 - kx1) * (ky2 - ky1)
        iou = inter / (a_c + a_k - inter + 1e-9)
        suppressed = jnp.any(iou > _THI)
        lm = (l100 == r) & (~suppressed) & (r < _MAXO) & valid
        x1_ref[0, pl.ds(ci, 1), :] = jnp.where(lm, cx1, kx1)
        y1_ref[0, pl.ds(ci, 1), :] = jnp.where(lm, cy1, ky1)
        x2_ref[0, pl.ds(ci, 1), :] = jnp.where(lm, cx2, kx2)
        y2_ref[0, pl.ds(ci, 1), :] = jnp.where(lm, cy2, ky2)
        sc_ref[0, pl.ds(ci, 1), :] = jnp.where(lm, mx, ks)
        return qs, counters

    jax.lax.fori_loop(0, n_cand, body,
                      (qs0, jnp.zeros((1, 128), f32)))


def kernel(conf, loc, anchor):
    b = conf.shape[0]
    f32 = jnp.float32
    outs = pl.pallas_call(
        _dpp_body,
        grid=(b,),
        in_specs=[
            pl.BlockSpec((1, _N, _CLS1), lambda i: (i, 0, 0)),
            pl.BlockSpec((1, _N // 8, 32), lambda i: (i, 0, 0)),
            pl.BlockSpec((_N // 8, 32), lambda i: (0, 0)),
        ],
        out_specs=[pl.BlockSpec((1, 80, _MAXO), lambda i: (i, 0, 0))] * 5,
        out_shape=[jax.ShapeDtypeStruct((b, 80, _MAXO), f32)] * 5,
        scratch_shapes=[
            pltpu.VMEM((_CH, 16), f32),
        ],
    )(conf, loc.reshape(b, _N // 8, 32), anchor.reshape(_N // 8, 32))
    return jnp.stack(outs, axis=-1)
